# initial kernel scaffold (unmeasured)
import jax
import jax.numpy as jnp
from jax import lax
from jax.experimental import pallas as pl
from jax.experimental.pallas import tpu as pltpu

N = 4
M = 4096
D = 4096
CH = M // N


def kernel(partial, resid, gamma):
    gamma2 = gamma.reshape(1, D)

    def body(partial_ref, resid_ref, gamma_ref, out_ref,
             rs_recv, acc, tmp,
             rs_send_sems, rs_recv_sems, ag_send_sems, ag_recv_sems,
             copy_sem):
        my_x = lax.axis_index("x")
        my_y = lax.axis_index("y")
        my_z = lax.axis_index("z")
        left = (my_x, lax.rem(my_y + N - 1, N), my_z)
        right = (my_x, lax.rem(my_y + 1, N), my_z)

        barrier = pltpu.get_barrier_semaphore()
        for nbr in (left, right):
            pl.semaphore_signal(barrier, inc=1, device_id=nbr,
                                device_id_type=pl.DeviceIdType.MESH)
        pl.semaphore_wait(barrier, 2)

        def rows(c):
            return pl.ds(c * CH, CH)

        for s in range(N - 1):
            c_send = lax.rem(my_y - s + 2 * N, N)
            c_recv = lax.rem(my_y - 1 - s + 2 * N, N)
            src = partial_ref.at[0, rows(c_send), :] if s == 0 else acc
            rdma = pltpu.make_async_remote_copy(
                src_ref=src,
                dst_ref=rs_recv.at[s],
                send_sem=rs_send_sems.at[s],
                recv_sem=rs_recv_sems.at[s],
                device_id=right,
                device_id_type=pl.DeviceIdType.MESH,
            )
            rdma.start()
            cp = pltpu.make_async_copy(
                partial_ref.at[0, rows(c_recv), :], tmp, copy_sem)
            cp.start()
            rdma.wait()
            cp.wait()
            acc[...] = rs_recv[s] + tmp[...]

        c_own = lax.rem(my_y + 1, N)
        cp = pltpu.make_async_copy(resid_ref.at[rows(c_own), :], tmp, copy_sem)
        cp.start()
        cp.wait()
        y = acc[...] + tmp[...]
        rms = jnp.sqrt(jnp.mean(y * y, axis=-1, keepdims=True) + 1e-6)
        acc[...] = y / rms * gamma_ref[...]

        cp = pltpu.make_async_copy(acc, out_ref.at[rows(c_own), :], copy_sem)
        cp.start()
        cp.wait()

        for s in range(N - 1):
            c_send = lax.rem(my_y + 1 - s + 2 * N, N)
            rdma = pltpu.make_async_remote_copy(
                src_ref=out_ref.at[rows(c_send), :],
                dst_ref=out_ref.at[rows(c_send), :],
                send_sem=ag_send_sems.at[s],
                recv_sem=ag_recv_sems.at[s],
                device_id=right,
                device_id_type=pl.DeviceIdType.MESH,
            )
            rdma.start()
            rdma.wait()

    return pl.pallas_call(
        body,
        out_shape=jax.ShapeDtypeStruct((M, D), jnp.float32),
        in_specs=[
            pl.BlockSpec(memory_space=pltpu.ANY),
            pl.BlockSpec(memory_space=pltpu.ANY),
            pl.BlockSpec(memory_space=pltpu.VMEM),
        ],
        out_specs=pl.BlockSpec(memory_space=pltpu.ANY),
        scratch_shapes=[
            pltpu.VMEM((N - 1, CH, D), jnp.float32),
            pltpu.VMEM((CH, D), jnp.float32),
            pltpu.VMEM((CH, D), jnp.float32),
            pltpu.SemaphoreType.DMA((N - 1,)),
            pltpu.SemaphoreType.DMA((N - 1,)),
            pltpu.SemaphoreType.DMA((N - 1,)),
            pltpu.SemaphoreType.DMA((N - 1,)),
            pltpu.SemaphoreType.DMA,
        ],
        compiler_params=pltpu.CompilerParams(collective_id=0),
    )(partial, resid, gamma2)


# baseline (device time: 1237407 ns/iter reference)
import jax
import jax.numpy as jnp
from jax import lax
from jax.experimental import pallas as pl
from jax.experimental.pallas import tpu as pltpu

N = 4
M = 4096
D = 4096
CH = M // N
T = 256
K = CH // T


def kernel(partial, resid, gamma):
    gamma2 = gamma.reshape(1, D)

    def body(partial_ref, resid_ref, gamma_ref,
             out_ref, rs_recv, acc_hbm,
             va, vb, vc,
             rs_send_sems, rs_recv_sems, ag_send_sems, ag_recv_sems,
             copy_sems):
        my_x = lax.axis_index("x")
        my_y = lax.axis_index("y")
        my_z = lax.axis_index("z")
        left = (my_x, lax.rem(my_y + N - 1, N), my_z)
        right = (my_x, lax.rem(my_y + 1, N), my_z)

        barrier = pltpu.get_barrier_semaphore()
        for nbr in (left, right):
            pl.semaphore_signal(barrier, inc=1, device_id=nbr,
                                device_id_type=pl.DeviceIdType.MESH)
        pl.semaphore_wait(barrier, 2)

        def rows(c, k=0):
            return pl.ds(c * CH + k * T, T) if k is not None else pl.ds(c * CH, CH)

        def chunk(c):
            return pl.ds(c * CH, CH)

        for s in range(N - 1):
            c_send = lax.rem(my_y - s + 2 * N, N)
            c_recv = lax.rem(my_y - 1 - s + 2 * N, N)
            src = partial_ref.at[0, chunk(c_send), :] if s == 0 else acc_hbm
            rdma = pltpu.make_async_remote_copy(
                src_ref=src,
                dst_ref=rs_recv.at[s],
                send_sem=rs_send_sems.at[s],
                recv_sem=rs_recv_sems.at[s],
                device_id=right,
                device_id_type=pl.DeviceIdType.MESH,
            )
            rdma.start()
            rdma.wait()
            for k in range(K):
                cp1 = pltpu.make_async_copy(
                    rs_recv.at[s, pl.ds(k * T, T), :], va, copy_sems.at[0])
                cp2 = pltpu.make_async_copy(
                    partial_ref.at[0, rows(c_recv, k), :], vb, copy_sems.at[1])
                cp1.start()
                cp2.start()
                cp1.wait()
                cp2.wait()
                vc[...] = va[...] + vb[...]
                cp3 = pltpu.make_async_copy(
                    vc, acc_hbm.at[pl.ds(k * T, T), :], copy_sems.at[2])
                cp3.start()
                cp3.wait()

        c_own = lax.rem(my_y + 1, N)
        for k in range(K):
            cp1 = pltpu.make_async_copy(
                acc_hbm.at[pl.ds(k * T, T), :], va, copy_sems.at[0])
            cp2 = pltpu.make_async_copy(
                resid_ref.at[rows(c_own, k), :], vb, copy_sems.at[1])
            cp1.start()
            cp2.start()
            cp1.wait()
            cp2.wait()
            y = va[...] + vb[...]
            rms = jnp.sqrt(jnp.mean(y * y, axis=-1, keepdims=True) + 1e-6)
            vc[...] = y / rms * gamma_ref[...]
            cp3 = pltpu.make_async_copy(
                vc, out_ref.at[rows(c_own, k), :], copy_sems.at[2])
            cp3.start()
            cp3.wait()

        for s in range(N - 1):
            c_send = lax.rem(my_y + 1 - s + 2 * N, N)
            rdma = pltpu.make_async_remote_copy(
                src_ref=out_ref.at[chunk(c_send), :],
                dst_ref=out_ref.at[chunk(c_send), :],
                send_sem=ag_send_sems.at[s],
                recv_sem=ag_recv_sems.at[s],
                device_id=right,
                device_id_type=pl.DeviceIdType.MESH,
            )
            rdma.start()
            rdma.wait()

    out, _, _ = pl.pallas_call(
        body,
        out_shape=(
            jax.ShapeDtypeStruct((M, D), jnp.float32),
            jax.ShapeDtypeStruct((N - 1, CH, D), jnp.float32),
            jax.ShapeDtypeStruct((CH, D), jnp.float32),
        ),
        in_specs=[
            pl.BlockSpec(memory_space=pl.ANY),
            pl.BlockSpec(memory_space=pl.ANY),
            pl.BlockSpec(memory_space=pltpu.VMEM),
        ],
        out_specs=(
            pl.BlockSpec(memory_space=pl.ANY),
            pl.BlockSpec(memory_space=pl.ANY),
            pl.BlockSpec(memory_space=pl.ANY),
        ),
        scratch_shapes=[
            pltpu.VMEM((T, D), jnp.float32),
            pltpu.VMEM((T, D), jnp.float32),
            pltpu.VMEM((T, D), jnp.float32),
            pltpu.SemaphoreType.DMA((N - 1,)),
            pltpu.SemaphoreType.DMA((N - 1,)),
            pltpu.SemaphoreType.DMA((N - 1,)),
            pltpu.SemaphoreType.DMA((N - 1,)),
            pltpu.SemaphoreType.DMA((3,)),
        ],
        compiler_params=pltpu.CompilerParams(collective_id=0),
    )(partial, resid, gamma2)
    return out


# device time: 1059161 ns/iter; 1.1683x vs baseline; 1.1683x over previous
import jax
import jax.numpy as jnp
from jax import lax
from jax.experimental import pallas as pl
from jax.experimental.pallas import tpu as pltpu

N = 4
M = 4096
D = 4096
HALF = M // 2
CH = HALF // N


def kernel(partial, resid, gamma):
    gamma2 = gamma.reshape(1, D)

    def body(partial_ref, resid_ref, gamma_ref, out_ref,
             rs_recv, acc, tmp,
             rs_send_sems, rs_recv_sems, ag_send_sems, ag_recv_sems,
             zx_send_sems, zx_recv_sems, copy_sem):
        my_x = lax.axis_index("x")
        my_y = lax.axis_index("y")
        my_z = lax.axis_index("z")
        flat = my_x * 16 + my_y * 4 + my_z
        left = flat + (lax.rem(my_y + N - 1, N) - my_y) * 4
        right = flat + (lax.rem(my_y + 1, N) - my_y) * 4
        partner = flat + jnp.where(my_z < 2, 2, -2)

        zh = my_z // 2
        base = zh * HALF
        obase = (1 - zh) * HALF

        barrier = pltpu.get_barrier_semaphore()
        for nbr in (left, right, partner):
            pl.semaphore_signal(barrier, inc=1, device_id=nbr,
                                device_id_type=pl.DeviceIdType.LOGICAL)
        pl.semaphore_wait(barrier, 3)

        def rows(b, c):
            return pl.ds(b + c * CH, CH)

        def chunk_idx(j):
            if j == 0:
                return lax.rem(my_y + 1, N)
            return lax.rem(my_y - (j - 1) + 2 * N, N)

        def produce(j):
            c = chunk_idx(j)
            rdma = pltpu.make_async_remote_copy(
                src_ref=out_ref.at[rows(base, c), :],
                dst_ref=out_ref.at[rows(base, c), :],
                send_sem=zx_send_sems.at[j],
                recv_sem=zx_recv_sems.at[j],
                device_id=partner,
                device_id_type=pl.DeviceIdType.LOGICAL,
            )
            rdma.start()

        for s in range(N - 1):
            c_send = lax.rem(my_y - s + 2 * N, N)
            c_recv = lax.rem(my_y - 1 - s + 2 * N, N)
            src = partial_ref.at[0, rows(base, c_send), :] if s == 0 else acc
            rdma = pltpu.make_async_remote_copy(
                src_ref=src,
                dst_ref=rs_recv.at[s],
                send_sem=rs_send_sems.at[s],
                recv_sem=rs_recv_sems.at[s],
                device_id=right,
                device_id_type=pl.DeviceIdType.LOGICAL,
            )
            rdma.start()
            cp = pltpu.make_async_copy(
                partial_ref.at[0, rows(base, c_recv), :], tmp, copy_sem)
            cp.start()
            rdma.wait()
            cp.wait()
            acc[...] = rs_recv[s] + tmp[...]

        c_own = lax.rem(my_y + 1, N)
        cp = pltpu.make_async_copy(resid_ref.at[rows(base, c_own), :], tmp,
                                   copy_sem)
        cp.start()
        cp.wait()
        y = acc[...] + tmp[...]
        rms = jnp.sqrt(jnp.mean(y * y, axis=-1, keepdims=True) + 1e-6)
        acc[...] = y / rms * gamma_ref[...]

        cp = pltpu.make_async_copy(acc, out_ref.at[rows(base, c_own), :],
                                   copy_sem)
        cp.start()
        cp.wait()
        produce(0)

        for s in range(N - 1):
            c_send = lax.rem(my_y + 1 - s + 2 * N, N)
            rdma = pltpu.make_async_remote_copy(
                src_ref=out_ref.at[rows(base, c_send), :],
                dst_ref=out_ref.at[rows(base, c_send), :],
                send_sem=ag_send_sems.at[s],
                recv_sem=ag_recv_sems.at[s],
                device_id=right,
                device_id_type=pl.DeviceIdType.LOGICAL,
            )
            rdma.start()
            rdma.wait()
            produce(s + 1)

        for j in range(N):
            c = chunk_idx(j)
            snd = pltpu.make_async_remote_copy(
                src_ref=out_ref.at[rows(base, c), :],
                dst_ref=out_ref.at[rows(base, c), :],
                send_sem=zx_send_sems.at[j],
                recv_sem=zx_recv_sems.at[j],
                device_id=partner,
                device_id_type=pl.DeviceIdType.LOGICAL,
            )
            snd.wait_send()
            recv = pltpu.make_async_remote_copy(
                src_ref=out_ref.at[rows(obase, c), :],
                dst_ref=out_ref.at[rows(obase, c), :],
                send_sem=zx_send_sems.at[j],
                recv_sem=zx_recv_sems.at[j],
                device_id=partner,
                device_id_type=pl.DeviceIdType.LOGICAL,
            )
            recv.wait_recv()

    return pl.pallas_call(
        body,
        out_shape=jax.ShapeDtypeStruct((M, D), jnp.float32),
        in_specs=[
            pl.BlockSpec(memory_space=pl.ANY),
            pl.BlockSpec(memory_space=pl.ANY),
            pl.BlockSpec(memory_space=pltpu.VMEM),
        ],
        out_specs=pl.BlockSpec(memory_space=pl.ANY),
        scratch_shapes=[
            pltpu.VMEM((N - 1, CH, D), jnp.float32),
            pltpu.VMEM((CH, D), jnp.float32),
            pltpu.VMEM((CH, D), jnp.float32),
            pltpu.SemaphoreType.DMA((N - 1,)),
            pltpu.SemaphoreType.DMA((N - 1,)),
            pltpu.SemaphoreType.DMA((N - 1,)),
            pltpu.SemaphoreType.DMA((N - 1,)),
            pltpu.SemaphoreType.DMA((N,)),
            pltpu.SemaphoreType.DMA((N,)),
            pltpu.SemaphoreType.DMA,
        ],
        compiler_params=pltpu.CompilerParams(
            collective_id=0,
            vmem_limit_bytes=60 * 1024 * 1024,
        ),
    )(partial, resid, gamma2)
